# baseline (speedup 1.0000x reference)
import functools
import math

import jax
import jax.numpy as jnp
from jax import lax
from jax.experimental import pallas as pl
from jax.experimental.pallas import tpu as pltpu
from jax.experimental.pallas import tpu_sc as plsc

D_INPUT = 100000
D_MODEL = 1024
BATCH = 4
SEQ = 4096
SCALE = math.sqrt(D_MODEL)

NC = 2
NS = 16
NW = NC * NS
L = 16

B_TOTAL = BATCH * SEQ
BPW = B_TOTAL // NW
CHUNK = 8
NCHUNK = BPW // CHUNK
VPR = D_MODEL // L
NBUF = 12
AHEAD = 10

_mesh = plsc.VectorSubcoreMesh(core_axis_name="c", subcore_axis_name="s")


@functools.partial(
    pl.kernel,
    out_type=jax.ShapeDtypeStruct((B_TOTAL, D_MODEL), jnp.float32),
    mesh=_mesh,
    scratch_types=[
        pltpu.VMEM((BPW,), jnp.int32),
        pltpu.VMEM((NBUF * CHUNK, D_MODEL), jnp.float32),
        pltpu.SemaphoreType.DMA((NBUF,)),
        pltpu.SemaphoreType.DMA((NBUF,)),
    ],
)
def _emb_lookup(x_hbm, lut_hbm, out_hbm, idx_v, ring, gsem, ssem):
    wid = lax.axis_index("s") * NC + lax.axis_index("c")
    base = wid * BPW

    pltpu.sync_copy(x_hbm.at[pl.ds(base, BPW)], idx_v)

    def gather_descr(j, p):
        return pltpu.make_async_copy(
            lut_hbm.at[idx_v.at[pl.ds(j * CHUNK, CHUNK)]],
            ring.at[pl.ds(p * CHUNK, CHUNK)],
            gsem.at[p])

    def store_descr(j, p):
        return pltpu.make_async_copy(
            ring.at[pl.ds(p * CHUNK, CHUNK)],
            out_hbm.at[pl.ds(base + j * CHUNK, CHUNK)],
            ssem.at[p])

    for j in range(AHEAD):
        gather_descr(j, j % NBUF).start()

    def step(j, _):
        p = lax.rem(j, NBUF)
        ja = j + AHEAD
        q = lax.rem(ja, NBUF)

        @pl.when(jnp.logical_and(ja < NCHUNK, ja >= NBUF))
        def _():
            store_descr(ja - NBUF, q).wait()

        @pl.when(ja < NCHUNK)
        def _():
            gather_descr(ja, q).start()

        gather_descr(j, p).wait()
        for r in range(CHUNK):
            for c in range(VPR):
                sl = pl.ds(c * L, L)
                ring[p * CHUNK + r, sl] = ring[p * CHUNK + r, sl] * SCALE
        store_descr(j, p).start()
        return 0

    lax.fori_loop(0, NCHUNK, step, 0, unroll=False)

    for i in range(NBUF):
        j = NCHUNK - NBUF + i
        store_descr(j, j % NBUF).wait()


def kernel(x, lut):
    out = _emb_lookup(x.reshape(B_TOTAL).astype(jnp.int32), lut)
    return out.reshape(BATCH, SEQ, D_MODEL)


# speedup vs baseline: 1.0214x; 1.0214x over previous
import functools
import math

import jax
import jax.numpy as jnp
from jax import lax
from jax.experimental import pallas as pl
from jax.experimental.pallas import tpu as pltpu
from jax.experimental.pallas import tpu_sc as plsc

D_INPUT = 100000
D_MODEL = 1024
BATCH = 4
SEQ = 4096
SCALE = math.sqrt(D_MODEL)

NC = 2
NS = 16
NW = NC * NS
L = 16

B_TOTAL = BATCH * SEQ
BPW = B_TOTAL // NW
CHUNK = 8
NCHUNK = BPW // CHUNK
VPR = D_MODEL // L
NBUF = 8
AHEAD = 6
HALF = CHUNK // 2

_mesh = plsc.VectorSubcoreMesh(core_axis_name="c", subcore_axis_name="s")


@functools.partial(
    pl.kernel,
    out_type=jax.ShapeDtypeStruct((B_TOTAL, D_MODEL), jnp.float32),
    mesh=_mesh,
    scratch_types=[
        pltpu.VMEM((BPW,), jnp.int32),
        pltpu.VMEM((NBUF * CHUNK, D_MODEL), jnp.float32),
        pltpu.SemaphoreType.DMA((NBUF,)),
        pltpu.SemaphoreType.DMA((NBUF,)),
    ],
)
def _emb_lookup(x_hbm, lut_hbm, out_hbm, idx_v, ring, gsem, ssem):
    wid = lax.axis_index("s") * NC + lax.axis_index("c")
    base = wid * BPW

    pltpu.sync_copy(x_hbm.at[pl.ds(base, BPW)], idx_v)

    def gather_descr(j, p):
        return pltpu.make_async_copy(
            lut_hbm.at[idx_v.at[pl.ds(j * CHUNK, CHUNK)]],
            ring.at[pl.ds(p * CHUNK, CHUNK)],
            gsem.at[p])

    def store_descr(j, p):
        return pltpu.make_async_copy(
            ring.at[pl.ds(p * CHUNK, CHUNK)],
            out_hbm.at[pl.ds(base + j * CHUNK, CHUNK)],
            ssem.at[p])

    for j in range(AHEAD):
        gather_descr(j, j % NBUF).start()

    def step(j, _):
        p = lax.rem(j, NBUF)
        ja = j + AHEAD
        q = lax.rem(ja, NBUF)

        @pl.when(jnp.logical_and(ja < NCHUNK, ja >= NBUF))
        def _():
            store_descr(ja - NBUF, q).wait()

        @pl.when(ja < NCHUNK)
        def _():
            gather_descr(ja, q).start()

        gather_descr(j, p).wait()
        for h in range(2):
            for r in range(h * HALF, (h + 1) * HALF):
                for c in range(VPR):
                    sl = pl.ds(c * L, L)
                    ring[p * CHUNK + r, sl] = ring[p * CHUNK + r, sl] * SCALE
            pltpu.make_async_copy(
                ring.at[pl.ds(p * CHUNK + h * HALF, HALF)],
                out_hbm.at[pl.ds(base + j * CHUNK + h * HALF, HALF)],
                ssem.at[p]).start()
        return 0

    lax.fori_loop(0, NCHUNK, step, 0, unroll=False)

    for i in range(NBUF):
        j = NCHUNK - NBUF + i
        store_descr(j, j % NBUF).wait()


def kernel(x, lut):
    out = _emb_lookup(x.reshape(B_TOTAL).astype(jnp.int32), lut)
    return out.reshape(BATCH, SEQ, D_MODEL)
